# Initial kernel scaffold; baseline (speedup 1.0000x reference)
import functools

import jax
import jax.numpy as jnp
from jax import lax
from jax.experimental import pallas as pl
from jax.experimental.pallas import tpu as pltpu
from jax.experimental.pallas import tpu_sc as plsc

N_NODES = 10000
N_EDGES = 320000
D_FEAT = 128
N_CLASS = 40

NUM_CORES = 2
NUM_SUBCORES = 16
NUM_TILES = NUM_CORES * NUM_SUBCORES
EDGES_PER_TILE = N_EDGES // NUM_TILES
CHUNK = 128
NCHUNK = -(-EDGES_PER_TILE // CHUNK)
PAD_EDGES = NCHUNK * CHUNK
ACC_ROWS = 10016
JUNK_ROW = N_NODES
ROWS_PER_SUBCORE = N_NODES // NUM_SUBCORES


def _sc_aggregate_body(src_hbm, dst_hbm, feat_hbm, zeros_hbm, out_hbm,
                       sidx_v, didx_v, rows_v, acc_s):
  cid = lax.axis_index("c")
  sid = lax.axis_index("s")
  wid = cid * NUM_SUBCORES + sid

  pltpu.sync_copy(src_hbm.at[wid], sidx_v)
  pltpu.sync_copy(dst_hbm.at[wid], didx_v)

  row0 = sid * ROWS_PER_SUBCORE

  @pl.when(cid == 0)
  def _():
    pltpu.sync_copy(feat_hbm.at[pl.ds(row0, ROWS_PER_SUBCORE)],
                    acc_s.at[pl.ds(row0, ROWS_PER_SUBCORE)])

  @pl.when(cid == 1)
  def _():
    pltpu.sync_copy(zeros_hbm.at[pl.ds(row0, ROWS_PER_SUBCORE)],
                    acc_s.at[pl.ds(row0, ROWS_PER_SUBCORE)])

  plsc.subcore_barrier()

  def body(j, carry):
    pltpu.sync_copy(feat_hbm.at[sidx_v.at[j]], rows_v)
    pltpu.sync_copy(rows_v, acc_s.at[didx_v.at[j]], add=True)
    return carry

  lax.fori_loop(0, NCHUNK, body, 0, unroll=False)

  plsc.subcore_barrier()

  pltpu.sync_copy(acc_s.at[pl.ds(row0, ROWS_PER_SUBCORE)],
                  out_hbm.at[cid, pl.ds(row0, ROWS_PER_SUBCORE)])


_sc_aggregate = functools.partial(
    pl.kernel,
    out_type=jax.ShapeDtypeStruct((NUM_CORES, N_NODES, D_FEAT), jnp.float32),
    mesh=plsc.VectorSubcoreMesh(core_axis_name="c", subcore_axis_name="s"),
    scratch_types=[
        pltpu.VMEM((NCHUNK, CHUNK), jnp.int32),
        pltpu.VMEM((NCHUNK, CHUNK), jnp.int32),
        pltpu.VMEM((CHUNK, D_FEAT), jnp.float32),
        pltpu.VMEM_SHARED((ACC_ROWS, D_FEAT), jnp.float32),
    ],
)(_sc_aggregate_body)


ROW_BLK = 1250


def _mlp1_body(p0_ref, p1_ref, x_ref, w_ref, b_ref, out_ref):
  a = p0_ref[...] + p1_ref[...] + x_ref[...]
  h = jnp.dot(a, w_ref[...], preferred_element_type=jnp.float32) + b_ref[...]
  out_ref[...] = jnp.maximum(h, 0.0)


def _mlp2_body(p0_ref, p1_ref, h1_ref, w2_ref, b2_ref, w3_ref, b3_ref,
               out_ref):
  a = p0_ref[...] + p1_ref[...] + h1_ref[...]
  h2 = jnp.dot(a, w2_ref[...], preferred_element_type=jnp.float32)
  h2 = jnp.maximum(h2 + b2_ref[...], 0.0)
  logits = jnp.dot(h2, w3_ref[...], preferred_element_type=jnp.float32)
  logits = logits + b3_ref[...]
  m = jnp.max(logits, axis=1, keepdims=True)
  lse = m + jnp.log(jnp.sum(jnp.exp(logits - m), axis=1, keepdims=True))
  out_ref[...] = logits - lse


def _row_block(i):
  return (i, 0)


def _full_block(i):
  return (0, 0)


_mlp1 = pl.pallas_call(
    _mlp1_body,
    grid=(N_NODES // ROW_BLK,),
    in_specs=[
        pl.BlockSpec((ROW_BLK, D_FEAT), _row_block),
        pl.BlockSpec((ROW_BLK, D_FEAT), _row_block),
        pl.BlockSpec((ROW_BLK, D_FEAT), _row_block),
        pl.BlockSpec((D_FEAT, D_FEAT), _full_block),
        pl.BlockSpec((1, D_FEAT), _full_block),
    ],
    out_specs=pl.BlockSpec((ROW_BLK, D_FEAT), _row_block),
    out_shape=jax.ShapeDtypeStruct((N_NODES, D_FEAT), jnp.float32),
)

_mlp2 = pl.pallas_call(
    _mlp2_body,
    grid=(N_NODES // ROW_BLK,),
    in_specs=[
        pl.BlockSpec((ROW_BLK, D_FEAT), _row_block),
        pl.BlockSpec((ROW_BLK, D_FEAT), _row_block),
        pl.BlockSpec((ROW_BLK, D_FEAT), _row_block),
        pl.BlockSpec((D_FEAT, D_FEAT), _full_block),
        pl.BlockSpec((1, D_FEAT), _full_block),
        pl.BlockSpec((D_FEAT, N_CLASS), _full_block),
        pl.BlockSpec((1, N_CLASS), _full_block),
    ],
    out_specs=pl.BlockSpec((ROW_BLK, N_CLASS), _row_block),
    out_shape=jax.ShapeDtypeStruct((N_NODES, N_CLASS), jnp.float32),
)


def _chunk_indices(idx, pad_value):
  per_tile = idx.reshape(NUM_TILES, EDGES_PER_TILE)
  padded = jnp.pad(per_tile, ((0, 0), (0, PAD_EDGES - EDGES_PER_TILE)),
                   constant_values=pad_value)
  return padded.reshape(NUM_TILES, NCHUNK, CHUNK)


@jax.jit
def kernel(x, edge_index, W1, b1, W2, b2, W3, b3):
  src = _chunk_indices(edge_index[0].astype(jnp.int32), 0)
  dst = _chunk_indices(edge_index[1].astype(jnp.int32), JUNK_ROW)
  zeros = jnp.zeros((N_NODES, D_FEAT), jnp.float32)

  p = _sc_aggregate(src, dst, x, zeros)
  h1 = _mlp1(p[0], p[1], x, W1, b1.reshape(1, D_FEAT))
  p2 = _sc_aggregate(src, dst, h1, zeros)
  return _mlp2(p2[0], p2[1], h1, W2, b2.reshape(1, D_FEAT),
               W3, b3.reshape(1, N_CLASS))


# speedup vs baseline: 5.1058x; 5.1058x over previous
import functools

import jax
import jax.numpy as jnp
from jax import lax
from jax.experimental import pallas as pl
from jax.experimental.pallas import tpu as pltpu
from jax.experimental.pallas import tpu_sc as plsc

N_NODES = 10000
N_EDGES = 320000
D_FEAT = 128
N_CLASS = 40

NUM_CORES = 2
NUM_SUBCORES = 16
NUM_TILES = NUM_CORES * NUM_SUBCORES
EDGES_PER_TILE = N_EDGES // NUM_TILES
CHUNK = 128
NCHUNK = -(-EDGES_PER_TILE // CHUNK)
PAD_EDGES = NCHUNK * CHUNK
ACC_ROWS = 10016
JUNK_ROW = N_NODES
ROWS_MAIN = 632
ROWS_TAIL = N_NODES - 15 * ROWS_MAIN


def _sc_aggregate_body(src_hbm, dst_hbm, feat_hbm, zeros_hbm, out_hbm,
                       sidx_v, didx_v, rows_v, acc_s):
  cid = lax.axis_index("c")
  sid = lax.axis_index("s")
  wid = cid * NUM_SUBCORES + sid

  pltpu.sync_copy(src_hbm.at[wid], sidx_v)
  pltpu.sync_copy(dst_hbm.at[wid], didx_v)

  row0 = sid * ROWS_MAIN

  def _init(nrows):
    @pl.when(cid == 0)
    def _():
      pltpu.sync_copy(feat_hbm.at[pl.ds(row0, nrows)],
                      acc_s.at[pl.ds(row0, nrows)])

    @pl.when(cid == 1)
    def _():
      pltpu.sync_copy(zeros_hbm.at[pl.ds(row0, nrows)],
                      acc_s.at[pl.ds(row0, nrows)])

  @pl.when(sid < NUM_SUBCORES - 1)
  def _():
    _init(ROWS_MAIN)

  @pl.when(sid == NUM_SUBCORES - 1)
  def _():
    _init(ROWS_TAIL)

  plsc.subcore_barrier()

  def body(j, carry):
    pltpu.sync_copy(feat_hbm.at[sidx_v.at[j]], rows_v)
    pltpu.sync_copy(rows_v, acc_s.at[didx_v.at[j]], add=True)
    return carry

  lax.fori_loop(0, NCHUNK, body, 0, unroll=False)

  plsc.subcore_barrier()

  @pl.when(sid < NUM_SUBCORES - 1)
  def _():
    pltpu.sync_copy(acc_s.at[pl.ds(row0, ROWS_MAIN)],
                    out_hbm.at[cid, pl.ds(row0, ROWS_MAIN)])

  @pl.when(sid == NUM_SUBCORES - 1)
  def _():
    pltpu.sync_copy(acc_s.at[pl.ds(row0, ROWS_TAIL)],
                    out_hbm.at[cid, pl.ds(row0, ROWS_TAIL)])


_sc_aggregate = functools.partial(
    pl.kernel,
    out_type=jax.ShapeDtypeStruct((NUM_CORES, N_NODES, D_FEAT), jnp.float32),
    mesh=plsc.VectorSubcoreMesh(core_axis_name="c", subcore_axis_name="s"),
    scratch_types=[
        pltpu.VMEM((NCHUNK, CHUNK), jnp.int32),
        pltpu.VMEM((NCHUNK, CHUNK), jnp.int32),
        pltpu.VMEM((CHUNK, D_FEAT), jnp.float32),
        pltpu.VMEM_SHARED((ACC_ROWS, D_FEAT), jnp.float32),
    ],
)(_sc_aggregate_body)


ROW_BLK = 2000


def _mlp1_body(p0_ref, p1_ref, w_ref, b_ref, out_ref):
  a = p0_ref[...] + p1_ref[...]
  h = jnp.dot(a, w_ref[...], preferred_element_type=jnp.float32) + b_ref[...]
  out_ref[...] = jnp.maximum(h, 0.0)


def _mlp2_body(p0_ref, p1_ref, w2_ref, b2_ref, w3_ref, b3_ref, out_ref):
  a = p0_ref[...] + p1_ref[...]
  h2 = jnp.dot(a, w2_ref[...], preferred_element_type=jnp.float32)
  h2 = jnp.maximum(h2 + b2_ref[...], 0.0)
  logits = jnp.dot(h2, w3_ref[...], preferred_element_type=jnp.float32)
  logits = logits + b3_ref[...]
  m = jnp.max(logits, axis=1, keepdims=True)
  lse = m + jnp.log(jnp.sum(jnp.exp(logits - m), axis=1, keepdims=True))
  out_ref[...] = logits - lse


def _row_block(i):
  return (i, 0)


def _full_block(i):
  return (0, 0)


_mlp1 = pl.pallas_call(
    _mlp1_body,
    grid=(N_NODES // ROW_BLK,),
    in_specs=[
        pl.BlockSpec((ROW_BLK, D_FEAT), _row_block),
        pl.BlockSpec((ROW_BLK, D_FEAT), _row_block),
        pl.BlockSpec((D_FEAT, D_FEAT), _full_block),
        pl.BlockSpec((1, D_FEAT), _full_block),
    ],
    out_specs=pl.BlockSpec((ROW_BLK, D_FEAT), _row_block),
    out_shape=jax.ShapeDtypeStruct((N_NODES, D_FEAT), jnp.float32),
)

_mlp2 = pl.pallas_call(
    _mlp2_body,
    grid=(N_NODES // ROW_BLK,),
    in_specs=[
        pl.BlockSpec((ROW_BLK, D_FEAT), _row_block),
        pl.BlockSpec((ROW_BLK, D_FEAT), _row_block),
        pl.BlockSpec((D_FEAT, D_FEAT), _full_block),
        pl.BlockSpec((1, D_FEAT), _full_block),
        pl.BlockSpec((D_FEAT, N_CLASS), _full_block),
        pl.BlockSpec((1, N_CLASS), _full_block),
    ],
    out_specs=pl.BlockSpec((ROW_BLK, N_CLASS), _row_block),
    out_shape=jax.ShapeDtypeStruct((N_NODES, N_CLASS), jnp.float32),
)


def _chunk_indices(idx, pad_value):
  per_tile = idx.reshape(NUM_TILES, EDGES_PER_TILE)
  padded = jnp.pad(per_tile, ((0, 0), (0, PAD_EDGES - EDGES_PER_TILE)),
                   constant_values=pad_value)
  return padded.reshape(NUM_TILES, NCHUNK, CHUNK)


@jax.jit
def kernel(x, edge_index, W1, b1, W2, b2, W3, b3):
  src = _chunk_indices(edge_index[0].astype(jnp.int32), 0)
  dst = _chunk_indices(edge_index[1].astype(jnp.int32), JUNK_ROW)
  zeros = jnp.zeros((N_NODES, D_FEAT), jnp.float32)

  p = _sc_aggregate(src, dst, x, zeros)
  h1 = _mlp1(p[0], p[1], W1, b1.reshape(1, D_FEAT))
  p2 = _sc_aggregate(src, dst, h1, zeros)
  return _mlp2(p2[0], p2[1], W2, b2.reshape(1, D_FEAT),
               W3, b3.reshape(1, N_CLASS))
